# baseline (speedup 1.0000x reference)
import functools

import jax
import jax.numpy as jnp
import numpy as np
from jax import lax
from jax.experimental import pallas as pl
from jax.experimental.pallas import tpu as pltpu

_K = 16
_NUM_HEADS = 16
_HEADS_PER_BLOCK = 2


def _knn_body(x_ref, out_ref):
    xb = x_ref[0]
    n = xb.shape[1]
    sq = jnp.sum(xb * xb, axis=0)
    xbb = xb.astype(jnp.bfloat16)
    prod = lax.dot_general(xbb, xbb, (((0,), (0,)), ((), ())),
                           preferred_element_type=jnp.float32)
    dist = sq[:, None] - 2.0 * prod + sq[None, :]
    col = lax.broadcasted_iota(jnp.int32, (n, n), 1)
    acc = jnp.zeros((n, n), jnp.float32)
    for _ in range(_K):
        m = jnp.min(dist, axis=1, keepdims=True)
        eq = dist == m
        first = jnp.min(jnp.where(eq, col, n), axis=1, keepdims=True)
        sel = col == first
        acc = acc + sel.astype(jnp.float32)
        dist = jnp.where(sel, jnp.float32(jnp.inf), dist)
    out_ref[0] = lax.dot_general(xb, acc, (((1,), (1,)), ((), ())),
                                 preferred_element_type=jnp.float32,
                                 precision=lax.Precision.HIGHEST) * (1.0 / _K)


def _knn_mean_t(x):
    b, c, n = x.shape
    return pl.pallas_call(
        _knn_body,
        grid=(b,),
        in_specs=[pl.BlockSpec((1, c, n), lambda i: (i, 0, 0))],
        out_specs=pl.BlockSpec((1, c, n), lambda i: (i, 0, 0)),
        out_shape=jax.ShapeDtypeStruct((b, c, n), jnp.float32),
    )(x)


def _mha_body(xq_ref, xe_ref, wq_ref, wk_ref, wv_ref, wo_ref,
              bq_ref, bk_ref, bv_ref, bo_ref, out_ref, *, dh):
    hp = pl.program_id(1)
    l = xq_ref.shape[1]
    dn = (((1,), (1,)), ((), ()))
    xq = xq_ref[0]
    xqb = xq.astype(jnp.bfloat16)
    xe = xe_ref[0]
    scale = 1.0 / np.sqrt(dh)
    q2 = ((lax.dot_general(xqb, wq_ref[...], dn,
                           preferred_element_type=jnp.float32) + bq_ref[0])
          * scale).astype(jnp.bfloat16)
    k2 = (lax.dot_general(xe, wk_ref[...], dn,
                          preferred_element_type=jnp.float32)
          + bk_ref[0]).astype(jnp.bfloat16)
    v2 = (lax.dot_general(xe, wv_ref[...], dn,
                          preferred_element_type=jnp.float32)
          + bv_ref[0]).astype(jnp.bfloat16)
    proj = None
    for h in range(_HEADS_PER_BLOCK):
        sl = slice(h * dh, (h + 1) * dh)
        s = lax.dot_general(q2[:, sl], k2[:, sl], dn,
                            preferred_element_type=jnp.float32)
        m = jnp.max(s, axis=1, keepdims=True)
        p = jnp.exp(s - m)
        a = (p / jnp.sum(p, axis=1, keepdims=True)).astype(jnp.bfloat16)
        oh = lax.dot_general(a, v2[:, sl], (((1,), (0,)), ((), ())),
                             preferred_element_type=jnp.float32)
        ph = lax.dot_general(oh.astype(jnp.bfloat16), wo_ref[:, sl], dn,
                             preferred_element_type=jnp.float32)
        proj = ph if proj is None else proj + ph

    @pl.when(hp == 0)
    def _():
        out_ref[0, :l, :] = xq
        out_ref[0, l:, :] = proj + bo_ref[0][None, :]

    @pl.when(hp != 0)
    def _():
        out_ref[0, l:, :] += proj


def kernel(x, x_enc, in_proj_weight, in_proj_bias, out_proj_weight, out_proj_bias):
    b, c, n = x.shape
    s, e = x_enc.shape[1], x_enc.shape[2]
    l = c
    dh = e // _NUM_HEADS
    hb = _HEADS_PER_BLOCK
    w = hb * dh
    nhp = _NUM_HEADS // hb

    xknn_t = _knn_mean_t(x)
    xq = jnp.stack([x, xknn_t], axis=3).reshape(b, c, 2 * n)

    xe = x_enc.astype(jnp.bfloat16)
    wq = in_proj_weight[:e].astype(jnp.bfloat16)
    wk = in_proj_weight[e:2 * e].astype(jnp.bfloat16)
    wv = in_proj_weight[2 * e:].astype(jnp.bfloat16)
    wo = out_proj_weight.astype(jnp.bfloat16)
    bq = in_proj_bias[:e].reshape(nhp, 1, w)
    bk = in_proj_bias[e:2 * e].reshape(nhp, 1, w)
    bv = in_proj_bias[2 * e:].reshape(nhp, 1, w)
    bo = out_proj_bias.reshape(1, e)

    grid = (b, nhp)
    out = pl.pallas_call(
        functools.partial(_mha_body, dh=dh),
        grid=grid,
        in_specs=[
            pl.BlockSpec((1, l, e), lambda i, j: (i, 0, 0)),
            pl.BlockSpec((1, s, e), lambda i, j: (i, 0, 0)),
            pl.BlockSpec((w, e), lambda i, j: (j, 0)),
            pl.BlockSpec((w, e), lambda i, j: (j, 0)),
            pl.BlockSpec((w, e), lambda i, j: (j, 0)),
            pl.BlockSpec((e, w), lambda i, j: (0, j)),
            pl.BlockSpec((1, 1, w), lambda i, j: (j, 0, 0)),
            pl.BlockSpec((1, 1, w), lambda i, j: (j, 0, 0)),
            pl.BlockSpec((1, 1, w), lambda i, j: (j, 0, 0)),
            pl.BlockSpec((1, e), lambda i, j: (0, 0)),
        ],
        out_specs=pl.BlockSpec((1, 2 * l, e), lambda i, j: (i, 0, 0)),
        out_shape=jax.ShapeDtypeStruct((b, 2 * l, e), jnp.float32),
        compiler_params=pltpu.CompilerParams(
            dimension_semantics=("parallel", "arbitrary"),
        ),
    )(xq, xe, wq, wk, wv, wo, bq, bk, bv, bo)
    return out
